# baseline (speedup 1.0000x reference)
import functools

import jax
import jax.numpy as jnp
from jax import lax
from jax.experimental import pallas as pl
from jax.experimental.pallas import tpu as pltpu
from jax.experimental.pallas import tpu_sc as plsc

N_, E_, T_ = 10000, 320000, 500000
AD, BD, AND_ = 128, 128, 64
TP = 524288
NW = 32
GROWS = TP // NW
IDXW = 128
CH = 1024
HALF = 512

SRANGE = 10000
NRANGE = 16
SPR = 10112
DUMP = 10000
STRIPE = SPR // 16
SROWS = TP // 16
SBLK = 2048

BT = 2000


def _mesh():
    return plsc.VectorSubcoreMesh(core_axis_name="c", subcore_axis_name="s")


def _m8(x):
    return pl.multiple_of(x, 8)


@functools.partial(
    pl.kernel,
    out_type=[jax.ShapeDtypeStruct((TP, BD), jnp.float32) for _ in range(3)],
    mesh=_mesh(),
    scratch_types=[
        pltpu.VMEM((CH // IDXW, IDXW), jnp.int32),
        pltpu.VMEM((HALF, BD), jnp.float32),
        pltpu.SemaphoreType.DMA,
    ],
)
def _gather3(vfeat, efeat, jidx2, kidx2, iidx2, center_o, bondi_o, bondj_o,
             idx_v, rows_v, sem):
    c = lax.axis_index("c")
    s = lax.axis_index("s")
    wid = s * 2 + c
    base = wid * GROWS
    nsub = CH // IDXW
    per_half = HALF // IDXW

    def body(ch, _):
        off = _m8(base + ch * CH)
        irow = _m8(off // IDXW)
        for idx_hbm, table, out_hbm in ((jidx2, vfeat, center_o),
                                        (kidx2, efeat, bondi_o),
                                        (iidx2, efeat, bondj_o)):
            pltpu.sync_copy(idx_hbm.at[pl.ds(irow, nsub)], idx_v)
            for h in range(CH // HALF):
                descs = []
                for p in range(per_half):
                    descs.append(pltpu.async_copy(
                        table.at[idx_v.at[h * per_half + p]],
                        rows_v.at[pl.ds(p * IDXW, IDXW)], sem))
                for d in descs:
                    d.wait()
                pltpu.sync_copy(rows_v,
                                out_hbm.at[pl.ds(_m8(off + h * HALF), HALF)])
        return ()

    lax.fori_loop(0, GROWS // CH, body, ())


@functools.partial(
    pl.kernel,
    out_type=jax.ShapeDtypeStruct((E_, BD), jnp.float32),
    mesh=_mesh(),
    scratch_types=[
        pltpu.VMEM_SHARED((SPR, BD), jnp.float32),
        pltpu.VMEM((SBLK // IDXW, IDXW), jnp.int32),
        pltpu.VMEM((IDXW,), jnp.int32),
        pltpu.VMEM((IDXW,), jnp.int32),
        pltpu.VMEM((IDXW, BD), jnp.float32),
        pltpu.VMEM((IDXW, BD), jnp.float32),
        pltpu.SemaphoreType.DMA,
        pltpu.SemaphoreType.DMA,
    ],
)
def _scatter(kidx2, upd, seg_o, spmem, kbuf, kdma0, kdma1, rows0, rows1,
             sem0, sem1):
    c = lax.axis_index("c")
    s = lax.axis_index("s")
    tbase = s * SROWS
    krows = SBLK // IDXW

    def zero_rows():
        def zi(i, _):
            for q in range(BD // 16):
                rows0[i, pl.ds(q * 16, 16)] = jnp.zeros((16,), jnp.float32)
            return ()
        lax.fori_loop(0, IDXW, zi, ())

    def range_body(r, _):
        lo = (c * NRANGE + r) * SRANGE
        zero_rows()
        for z in range(STRIPE // IDXW):
            pltpu.sync_copy(rows0, spmem.at[pl.ds(_m8(s * STRIPE + z * IDXW),
                                                  IDXW)])
        tail = STRIPE % IDXW
        pltpu.sync_copy(
            rows0.at[pl.ds(0, tail)],
            spmem.at[pl.ds(_m8(s * STRIPE + STRIPE - tail), tail)])
        plsc.subcore_barrier()

        def pair_body(i, _2):
            @pl.when(lax.rem(i, 8) == 0)
            def _():
                blk_off = _m8(tbase + (i // 8) * SBLK)
                pltpu.sync_copy(kidx2.at[pl.ds(_m8(blk_off // IDXW), krows)],
                                kbuf)
            for b in range(2):
                kd = (kdma0, kdma1)[b]
                rw = (rows0, rows1)[b]
                sem = (sem0, sem1)[b]
                sub = i * 2 + b
                row = lax.rem(sub, krows)

                @pl.when(i > 0)
                def _(kd=kd, rw=rw, sem=sem):
                    pltpu.make_async_copy(rw, spmem.at[kd], sem).wait()
                for q in range(IDXW // 16):
                    kv = kbuf[row, pl.ds(q * 16, 16)]
                    inr = (kv >= lo) & (kv < lo + SRANGE)
                    kd[pl.ds(q * 16, 16)] = jnp.where(inr, kv - lo, DUMP)
                pltpu.sync_copy(upd.at[pl.ds(_m8(tbase + sub * IDXW), IDXW)],
                                rw)
                pltpu.async_copy(rw, spmem.at[kd], sem, add=True)
            return ()
        lax.fori_loop(0, SROWS // IDXW // 2, pair_body, ())
        pltpu.make_async_copy(rows0, spmem.at[kdma0], sem0).wait()
        pltpu.make_async_copy(rows1, spmem.at[kdma1], sem1).wait()
        plsc.subcore_barrier()

        obase = _m8(s * STRIPE)
        @pl.when(s < 15)
        def _():
            pltpu.sync_copy(spmem.at[pl.ds(obase, STRIPE)],
                            seg_o.at[pl.ds(_m8(lo + obase), STRIPE)])
        @pl.when(s == 15)
        def _():
            last = SRANGE - 15 * STRIPE
            pltpu.sync_copy(spmem.at[pl.ds(obase, last)],
                            seg_o.at[pl.ds(_m8(lo + obase), last)])
        plsc.subcore_barrier()
        return ()

    lax.fori_loop(0, NRANGE, range_body, ())


def _mm_stats_body(c_ref, bi_ref, bj_ref, a_ref, wc_ref, wg_ref,
                   core_ref, gate_ref, stats_ref):
    i = pl.program_id(0)
    total = jnp.concatenate(
        [c_ref[...], bi_ref[...], bj_ref[...], a_ref[...]], axis=1)
    core = jnp.dot(total, wc_ref[...], preferred_element_type=jnp.float32)
    gate = jnp.dot(total, wg_ref[...], preferred_element_type=jnp.float32)
    core_ref[...] = core
    gate_ref[...] = gate

    @pl.when(i == 0)
    def _():
        stats_ref[...] = jnp.zeros_like(stats_ref)

    z = jnp.zeros((BD,), jnp.float32)
    stats_ref[...] += jnp.stack(
        [core.sum(0), (core * core).sum(0),
         gate.sum(0), (gate * gate).sum(0), z, z, z, z])


def _mm_stats(center, bondi, bondj, angle, wc, wg):
    grid = T_ // BT
    return pl.pallas_call(
        _mm_stats_body,
        grid=(grid,),
        in_specs=[
            pl.BlockSpec((BT, BD), lambda i: (i, 0)),
            pl.BlockSpec((BT, BD), lambda i: (i, 0)),
            pl.BlockSpec((BT, BD), lambda i: (i, 0)),
            pl.BlockSpec((BT, AND_), lambda i: (i, 0)),
            pl.BlockSpec((AD + 2 * BD + AND_, BD), lambda i: (0, 0)),
            pl.BlockSpec((AD + 2 * BD + AND_, BD), lambda i: (0, 0)),
        ],
        out_specs=[
            pl.BlockSpec((BT, BD), lambda i: (i, 0)),
            pl.BlockSpec((BT, BD), lambda i: (i, 0)),
            pl.BlockSpec((8, BD), lambda i: (0, 0)),
        ],
        out_shape=[
            jax.ShapeDtypeStruct((TP, BD), jnp.float32),
            jax.ShapeDtypeStruct((TP, BD), jnp.float32),
            jax.ShapeDtypeStruct((8, BD), jnp.float32),
        ],
    )(center, bondi, bondj, angle, wc, wg)


def _act_body(core_ref, gate_ref, p_ref, upd_ref):
    p = p_ref[...]
    x = core_ref[...] * p[0:1, :] + p[1:2, :]
    y = gate_ref[...] * p[2:3, :] + p[3:4, :]
    upd_ref[...] = x * jax.nn.sigmoid(x) * jax.nn.sigmoid(y)


def _act(core, gate, params):
    grid = T_ // BT
    return pl.pallas_call(
        _act_body,
        grid=(grid,),
        in_specs=[
            pl.BlockSpec((BT, BD), lambda i: (i, 0)),
            pl.BlockSpec((BT, BD), lambda i: (i, 0)),
            pl.BlockSpec((8, BD), lambda i: (0, 0)),
        ],
        out_specs=pl.BlockSpec((BT, BD), lambda i: (i, 0)),
        out_shape=jax.ShapeDtypeStruct((TP, BD), jnp.float32),
    )(core, gate, params)


def _final_body(seg_ref, ef_ref, w_ref, out_ref):
    out_ref[...] = jnp.dot(seg_ref[...], w_ref[...],
                           preferred_element_type=jnp.float32) + ef_ref[...]


def _final(seg, edge_feat, w_out):
    grid = E_ // BT
    return pl.pallas_call(
        _final_body,
        grid=(grid,),
        in_specs=[
            pl.BlockSpec((BT, BD), lambda i: (i, 0)),
            pl.BlockSpec((BT, BD), lambda i: (i, 0)),
            pl.BlockSpec((BD, BD), lambda i: (0, 0)),
        ],
        out_specs=pl.BlockSpec((BT, BD), lambda i: (i, 0)),
        out_shape=jax.ShapeDtypeStruct((E_, BD), jnp.float32),
    )(seg, edge_feat, w_out)


@jax.jit
def kernel(vertex_feat, edge_feat, angle_feat, edge_index, k_idx, j_idx, i_idx,
           W_core, W_gate, bn_core_gamma, bn_core_beta,
           bn_gate_gamma, bn_gate_beta, W_out):
    pad = TP - T_
    k32 = k_idx.astype(jnp.int32)
    jpad2 = jnp.concatenate(
        [j_idx.astype(jnp.int32), jnp.zeros((pad,), jnp.int32)]).reshape(-1, IDXW)
    ipad2 = jnp.concatenate(
        [i_idx.astype(jnp.int32), jnp.zeros((pad,), jnp.int32)]).reshape(-1, IDXW)
    kpadg2 = jnp.concatenate(
        [k32, jnp.zeros((pad,), jnp.int32)]).reshape(-1, IDXW)
    kpads2 = jnp.concatenate(
        [k32, jnp.full((pad,), E_, jnp.int32)]).reshape(-1, IDXW)

    center, bondi, bondj = _gather3(vertex_feat, edge_feat, jpad2, kpadg2, ipad2)
    core, gate, stats = _mm_stats(center, bondi, bondj, angle_feat,
                                  W_core, W_gate)

    inv_t = jnp.float32(1.0 / T_)
    mean_c = stats[0] * inv_t
    var_c = stats[1] * inv_t - mean_c * mean_c
    mean_g = stats[2] * inv_t
    var_g = stats[3] * inv_t - mean_g * mean_g
    sc = bn_core_gamma / jnp.sqrt(var_c + 1e-5)
    sg = bn_gate_gamma / jnp.sqrt(var_g + 1e-5)
    params = jnp.stack([
        sc, bn_core_beta - mean_c * sc,
        sg, bn_gate_beta - mean_g * sg,
        jnp.zeros_like(sc), jnp.zeros_like(sc),
        jnp.zeros_like(sc), jnp.zeros_like(sc)])

    upd = _act(core, gate, params)
    seg = _scatter(kpads2, upd)
    return _final(seg, edge_feat, W_out)
